# baseline (speedup 1.0000x reference)
import functools

import jax
import jax.numpy as jnp
from jax import lax
from jax.experimental import pallas as pl
from jax.experimental.pallas import tpu as pltpu
from jax.experimental.pallas import tpu_sc as plsc

_S = 2048
_H = 16
_NB = 32
_MAXD = 128
_NW = 32
_RPW = _S // _NW
_L = 16


def _sc_bias_body(pos_hbm, ttf_hbm, out_hbm, pos_v, tt_v, row_v):
    wid = lax.axis_index("s") * 2 + lax.axis_index("c")
    base = wid * _RPW
    pltpu.sync_copy(pos_hbm, pos_v)
    pltpu.sync_copy(ttf_hbm, tt_v)

    def row_loop(r, carry):
        i = base + r
        pib = plsc.load_gather(pos_v, [jnp.full((_L,), i, jnp.int32)])

        def chunk(c, carry2):
            pj = pos_v[pl.ds(c * _L, _L)]
            d = jnp.clip(pib - pj, -_MAXD, _MAXD) + _MAXD
            b = jnp.minimum(lax.shift_right_arithmetic(d, 3), _NB - 1)
            for h in range(_H):
                row_v[h, pl.ds(c * _L, _L)] = plsc.load_gather(
                    tt_v, [b + h * _NB])
            return carry2

        lax.fori_loop(0, _S // _L, chunk, 0)
        pltpu.sync_copy(row_v, out_hbm.at[:, i, :])
        return carry

    lax.fori_loop(0, _RPW, row_loop, 0)


@jax.jit
def kernel(positions, scale, table):
    ttf = (table * scale[:, 0, 0][None, :]).T.reshape(-1)
    pos = positions.reshape(-1)
    mesh = plsc.VectorSubcoreMesh(core_axis_name="c", subcore_axis_name="s")
    sc_call = pl.kernel(
        _sc_bias_body,
        out_type=jax.ShapeDtypeStruct((_H, _S, _S), jnp.float32),
        mesh=mesh,
        scratch_types=[
            pltpu.VMEM((_S,), jnp.int32),
            pltpu.VMEM((_H * _NB,), jnp.float32),
            pltpu.VMEM((_H, _S), jnp.float32),
        ],
        compiler_params=pltpu.CompilerParams(needs_layout_passes=False),
    )
    out = sc_call(pos, ttf)
    return out[None]


# speedup vs baseline: 1.2684x; 1.2684x over previous
import functools

import jax
import jax.numpy as jnp
from jax import lax
from jax.experimental import pallas as pl
from jax.experimental.pallas import tpu as pltpu
from jax.experimental.pallas import tpu_sc as plsc

_S = 2048
_H = 16
_NB = 32
_MAXD = 128
_NW = 32
_RPW = _S // _NW
_L = 16


def _sc_bias_body(pos_hbm, ttf_hbm, out_hbm, pos_v, tt_v, row_v,
                  sem0, sem1):
    wid = lax.axis_index("s") * 2 + lax.axis_index("c")
    base = wid * _RPW
    pltpu.sync_copy(pos_hbm, pos_v)
    pltpu.sync_copy(ttf_hbm, tt_v)
    sems = (sem0, sem1)

    def compute_row(i, buf):
        pib = plsc.load_gather(pos_v, [jnp.full((_L,), i, jnp.int32)])

        def chunk(c, carry2):
            pj = pos_v[pl.ds(c * _L, _L)]
            d = jnp.clip(pib - pj, -_MAXD, _MAXD) + _MAXD
            b = jnp.minimum(lax.shift_right_arithmetic(d, 3), _NB - 1)
            for h in range(_H):
                row_v[buf, h, pl.ds(c * _L, _L)] = plsc.load_gather(
                    tt_v, [b + h * _NB])
            return carry2

        lax.fori_loop(0, _S // _L, chunk, 0, unroll=4)

    def pair_loop(g, carry):
        for b in range(2):
            i = base + g * 2 + b

            @pl.when(g > 0)
            def _wait():
                pltpu.make_async_copy(
                    row_v.at[b], out_hbm.at[:, i - 2, :], sems[b]).wait()

            compute_row(i, b)
            pltpu.async_copy(row_v.at[b], out_hbm.at[:, i, :], sems[b])
        return carry

    lax.fori_loop(0, _RPW // 2, pair_loop, 0)
    for b in range(2):
        pltpu.make_async_copy(
            row_v.at[b], out_hbm.at[:, base + _RPW - 2 + b, :],
            sems[b]).wait()


@jax.jit
def kernel(positions, scale, table):
    ttf = (table * scale[:, 0, 0][None, :]).T.reshape(-1)
    pos = positions.reshape(-1)
    mesh = plsc.VectorSubcoreMesh(core_axis_name="c", subcore_axis_name="s")
    sc_call = pl.kernel(
        _sc_bias_body,
        out_type=jax.ShapeDtypeStruct((_H, _S, _S), jnp.float32),
        mesh=mesh,
        scratch_types=[
            pltpu.VMEM((_S,), jnp.int32),
            pltpu.VMEM((_H * _NB,), jnp.float32),
            pltpu.VMEM((2, _H, _S), jnp.float32),
            pltpu.SemaphoreType.DMA,
            pltpu.SemaphoreType.DMA,
        ],
        compiler_params=pltpu.CompilerParams(needs_layout_passes=False),
    )
    out = sc_call(pos, ttf)
    return out[None]


# speedup vs baseline: 4.4532x; 3.5110x over previous
import functools

import jax
import jax.numpy as jnp
from jax import lax
from jax.experimental import pallas as pl
from jax.experimental.pallas import tpu as pltpu
from jax.experimental.pallas import tpu_sc as plsc

_S = 2048
_H = 16
_NB = 32
_MAXD = 128
_NW = 32
_RPW = _S // _NW
_L = 16


def _sc_bias_body(pos_hbm, ttf_hbm, out_hbm, pos_v, tt_v, row_v,
                  sem0, sem1):
    wid = lax.axis_index("s") * 2 + lax.axis_index("c")
    base = wid * _RPW
    pltpu.sync_copy(pos_hbm, pos_v)
    pltpu.sync_copy(ttf_hbm, tt_v)
    sems = (sem0, sem1)

    def compute_row(i, buf):
        pib = plsc.load_gather(pos_v, [jnp.full((_L,), i, jnp.int32)])

        @plsc.parallel_loop(0, _S // _L, unroll=4)
        def chunk(c):
            pj = pos_v[pl.ds(c * _L, _L)]
            d = jnp.clip(pib - pj, -_MAXD, _MAXD) + _MAXD
            b = jnp.minimum(lax.shift_right_arithmetic(d, 3), _NB - 1)
            for h in range(_H):
                row_v[buf, h, pl.ds(c * _L, _L)] = plsc.load_gather(
                    tt_v, [b + h * _NB])

    def pair_loop(g, carry):
        for b in range(2):
            i = base + g * 2 + b

            @pl.when(g > 0)
            def _wait():
                pltpu.make_async_copy(
                    row_v.at[b], out_hbm.at[:, i - 2, :], sems[b]).wait()

            compute_row(i, b)
            pltpu.async_copy(row_v.at[b], out_hbm.at[:, i, :], sems[b])
        return carry

    lax.fori_loop(0, _RPW // 2, pair_loop, 0)
    for b in range(2):
        pltpu.make_async_copy(
            row_v.at[b], out_hbm.at[:, base + _RPW - 2 + b, :],
            sems[b]).wait()


@jax.jit
def kernel(positions, scale, table):
    ttf = (table * scale[:, 0, 0][None, :]).T.reshape(-1)
    pos = positions.reshape(-1)
    mesh = plsc.VectorSubcoreMesh(core_axis_name="c", subcore_axis_name="s")
    sc_call = pl.kernel(
        _sc_bias_body,
        out_type=jax.ShapeDtypeStruct((_H, _S, _S), jnp.float32),
        mesh=mesh,
        scratch_types=[
            pltpu.VMEM((_S,), jnp.int32),
            pltpu.VMEM((_H * _NB,), jnp.float32),
            pltpu.VMEM((2, _H, _S), jnp.float32),
            pltpu.SemaphoreType.DMA,
            pltpu.SemaphoreType.DMA,
        ],
        compiler_params=pltpu.CompilerParams(needs_layout_passes=False),
    )
    out = sc_call(pos, ttf)
    return out[None]
